# baseline (speedup 1.0000x reference)
import functools

import jax
import jax.numpy as jnp
from jax import lax
from jax.experimental import pallas as pl
from jax.experimental.pallas import tpu as pltpu
from jax.experimental.pallas import tpu_sc as plsc

B, L, V, D, H, C = 16384, 50, 100000, 128, 256, 1000
CP = 1024
LANES = 16
NC, NS = 2, 16
NW = NC * NS
RPW = B // NW
RG = 8
NG = RPW // RG
DG = D // LANES
JU = 5


def _pool_sc(x, emb):
    mesh = plsc.VectorSubcoreMesh(core_axis_name="c", subcore_axis_name="s")

    @functools.partial(
        pl.kernel,
        out_type=jax.ShapeDtypeStruct((B, D), jnp.float32),
        mesh=mesh,
        scratch_types=[
            pltpu.VMEM((2, RG, L), jnp.int32),
            pltpu.VMEM((2, RG, L, D), jnp.float32),
            pltpu.VMEM((2, RG, D), jnp.float32),
            pltpu.SemaphoreType.DMA,
            pltpu.SemaphoreType.DMA,
            pltpu.SemaphoreType.DMA,
            pltpu.SemaphoreType.DMA,
            pltpu.SemaphoreType.DMA,
        ],
    )
    def pool(x_hbm, emb_hbm, out_hbm, idx_v, rows_v, out_v,
             sem0, sem1, isem, osem0, osem1):
        sems = (sem0, sem1)
        osems = (osem0, osem1)
        wid = lax.axis_index("s") * NC + lax.axis_index("c")
        row0 = wid * RPW

        def fire_idx(slot, g):
            r0 = row0 + g * RG
            pltpu.async_copy(x_hbm.at[pl.ds(r0, RG), :], idx_v.at[slot], isem)

        def wait_idx(slot, g):
            r0 = row0 + g * RG
            pltpu.make_async_copy(
                x_hbm.at[pl.ds(r0, RG), :], idx_v.at[slot], isem).wait()

        def fire_gather(slot, g):
            wait_idx(slot, g)
            for r in range(RG):
                pltpu.async_copy(
                    emb_hbm.at[idx_v.at[slot, r]], rows_v.at[slot, r], sems[slot]
                )

        def drain(slot):
            for r in range(RG):
                pltpu.make_async_copy(
                    emb_hbm.at[idx_v.at[slot, r]], rows_v.at[slot, r], sems[slot]
                ).wait()

        def wait_out(slot):
            pltpu.make_async_copy(
                out_v.at[slot], out_hbm.at[pl.ds(row0, RG), :], osems[slot]
            ).wait()

        def reduce(slot, g):
            for r in range(RG):
                def body(jj, accs):
                    accs = list(accs)
                    for u in range(JU):
                        j = jj * JU + u
                        for d in range(DG):
                            accs[d] = accs[d] + rows_v[
                                slot, r, j, pl.ds(d * LANES, LANES)]
                    return tuple(accs)
                accs = lax.fori_loop(
                    0, L // JU, body,
                    tuple(jnp.zeros((LANES,), jnp.float32) for _ in range(DG)),
                )
                for d in range(DG):
                    out_v[slot, r, pl.ds(d * LANES, LANES)] = accs[d]
            pltpu.async_copy(
                out_v.at[slot], out_hbm.at[pl.ds(row0 + g * RG, RG), :],
                osems[slot])

        fire_idx(0, 0)
        fire_gather(0, 0)
        fire_idx(1, 1)
        fire_gather(1, 1)
        drain(0)
        fire_idx(0, 2)
        reduce(0, 0)
        fire_gather(0, 2)
        drain(1)
        fire_idx(1, 3)
        reduce(1, 1)

        def body(k, carry):
            g = 2 * k
            fire_gather(1, g + 1)
            drain(0)
            fire_idx(0, g + 2)
            wait_out(0)
            reduce(0, g)
            fire_gather(0, g + 2)
            drain(1)
            fire_idx(1, g + 3)
            wait_out(1)
            reduce(1, g + 1)
            return carry

        lax.fori_loop(1, NG // 2 - 1, body, 0)
        fire_gather(1, NG - 1)
        drain(0)
        wait_out(0)
        reduce(0, NG - 2)
        drain(1)
        wait_out(1)
        reduce(1, NG - 1)
        wait_out(0)
        wait_out(1)

    return pool(x, emb)


def _mlp_tc(pooled, w1, b1, w2p, b2p):
    BM = 1024

    def body(p_ref, w1_ref, b1_ref, w2_ref, b2_ref, o_ref):
        h = jnp.dot(p_ref[...], w1_ref[...], preferred_element_type=jnp.float32)
        h = jnp.maximum(h + b1_ref[...], 0.0)
        logits = jnp.dot(h, w2_ref[...], preferred_element_type=jnp.float32)
        logits = logits + b2_ref[...]
        m = jnp.max(logits, axis=1, keepdims=True)
        z = logits - m
        o_ref[...] = z - jnp.log(jnp.sum(jnp.exp(z), axis=1, keepdims=True))

    return pl.pallas_call(
        body,
        grid=(B // BM,),
        in_specs=[
            pl.BlockSpec((BM, D), lambda i: (i, 0)),
            pl.BlockSpec((D, H), lambda i: (0, 0)),
            pl.BlockSpec((1, H), lambda i: (0, 0)),
            pl.BlockSpec((H, CP), lambda i: (0, 0)),
            pl.BlockSpec((1, CP), lambda i: (0, 0)),
        ],
        out_specs=pl.BlockSpec((BM, CP), lambda i: (i, 0)),
        out_shape=jax.ShapeDtypeStruct((B, CP), jnp.float32),
    )(pooled, w1, b1.reshape(1, H), w2p, b2p.reshape(1, CP))


def kernel(x, emb, w1, b1, w2, b2):
    sums = _pool_sc(x.astype(jnp.int32), emb)
    w1s = w1 * (1.0 / L)
    w2p = jnp.pad(w2, ((0, 0), (0, CP - C)))
    b2p = jnp.concatenate([b2, jnp.full((CP - C,), -1e30, jnp.float32)])
    out = _mlp_tc(sums, w1s, b1, w2p, b2p)
    return out[:, :C]


# speedup vs baseline: 1.0228x; 1.0228x over previous
import functools

import jax
import jax.numpy as jnp
from jax import lax
from jax.experimental import pallas as pl
from jax.experimental.pallas import tpu as pltpu
from jax.experimental.pallas import tpu_sc as plsc

B, L, V, D, H, C = 16384, 50, 100000, 128, 256, 1000
CP = 1024
LANES = 16
NC, NS = 2, 16
NW = NC * NS
RPW = B // NW
RG = 8
NG = RPW // RG
DG = D // LANES
JU = 5


def _pool_sc(x, emb):
    mesh = plsc.VectorSubcoreMesh(core_axis_name="c", subcore_axis_name="s")

    @functools.partial(
        pl.kernel,
        out_type=jax.ShapeDtypeStruct((B, D), jnp.float32),
        mesh=mesh,
        scratch_types=[
            pltpu.VMEM((2, RG, L), jnp.int32),
            pltpu.VMEM((2, RG, L, D), jnp.float32),
            pltpu.VMEM((RG, D), jnp.float32),
            pltpu.SemaphoreType.DMA,
            pltpu.SemaphoreType.DMA,
            pltpu.SemaphoreType.DMA,
        ],
    )
    def pool(x_hbm, emb_hbm, out_hbm, idx_v, rows_v, out_v, sem0, sem1, isem):
        sems = (sem0, sem1)
        wid = lax.axis_index("s") * NC + lax.axis_index("c")
        row0 = wid * RPW

        def fire_idx(slot, g):
            r0 = row0 + g * RG
            pltpu.async_copy(x_hbm.at[pl.ds(r0, RG), :], idx_v.at[slot], isem)

        def wait_idx(slot, g):
            r0 = row0 + g * RG
            pltpu.make_async_copy(
                x_hbm.at[pl.ds(r0, RG), :], idx_v.at[slot], isem).wait()

        def fire_gather(slot, g):
            wait_idx(slot, g)
            for r in range(RG):
                pltpu.async_copy(
                    emb_hbm.at[idx_v.at[slot, r]], rows_v.at[slot, r], sems[slot]
                )

        def drain(slot):
            for r in range(RG):
                pltpu.make_async_copy(
                    emb_hbm.at[idx_v.at[slot, r]], rows_v.at[slot, r], sems[slot]
                ).wait()

        def reduce(slot, g):
            for r in range(RG):
                def body(jj, accs):
                    accs = list(accs)
                    for u in range(JU):
                        j = jj * JU + u
                        for d in range(DG):
                            accs[d] = accs[d] + rows_v[
                                slot, r, j, pl.ds(d * LANES, LANES)]
                    return tuple(accs)
                accs = lax.fori_loop(
                    0, L // JU, body,
                    tuple(jnp.zeros((LANES,), jnp.float32) for _ in range(DG)),
                )
                for d in range(DG):
                    out_v[r, pl.ds(d * LANES, LANES)] = accs[d]
            pltpu.sync_copy(out_v, out_hbm.at[pl.ds(row0 + g * RG, RG), :])

        fire_idx(0, 0)
        fire_gather(0, 0)
        fire_idx(1, 1)

        def body(k, carry):
            g = 2 * k
            fire_gather(1, g + 1)
            drain(0)
            fire_idx(0, g + 2)
            reduce(0, g)
            fire_gather(0, g + 2)
            drain(1)
            fire_idx(1, g + 3)
            reduce(1, g + 1)
            return carry

        lax.fori_loop(0, NG // 2 - 1, body, 0)
        fire_gather(1, NG - 1)
        drain(0)
        reduce(0, NG - 2)
        drain(1)
        reduce(1, NG - 1)

    return pool(x, emb)


def _mlp_tc(pooled, w1, b1, w2p, b2p):
    BM = 1024

    def body(p_ref, w1_ref, b1_ref, w2_ref, b2_ref, o_ref):
        h = jnp.dot(p_ref[...], w1_ref[...], preferred_element_type=jnp.float32)
        h = jnp.maximum(h + b1_ref[...], 0.0)
        logits = jnp.dot(h, w2_ref[...], preferred_element_type=jnp.float32)
        logits = logits + b2_ref[...]
        m = jnp.max(logits, axis=1, keepdims=True)
        z = logits - m
        o_ref[...] = z - jnp.log(jnp.sum(jnp.exp(z), axis=1, keepdims=True))

    return pl.pallas_call(
        body,
        grid=(B // BM,),
        in_specs=[
            pl.BlockSpec((BM, D), lambda i: (i, 0)),
            pl.BlockSpec((D, H), lambda i: (0, 0)),
            pl.BlockSpec((1, H), lambda i: (0, 0)),
            pl.BlockSpec((H, CP), lambda i: (0, 0)),
            pl.BlockSpec((1, CP), lambda i: (0, 0)),
        ],
        out_specs=pl.BlockSpec((BM, CP), lambda i: (i, 0)),
        out_shape=jax.ShapeDtypeStruct((B, CP), jnp.float32),
    )(pooled, w1, b1.reshape(1, H), w2p, b2p.reshape(1, CP))


def kernel(x, emb, w1, b1, w2, b2):
    sums = _pool_sc(x.astype(jnp.int32), emb)
    w1s = w1 * (1.0 / L)
    w2p = jnp.pad(w2, ((0, 0), (0, CP - C)))
    b2p = jnp.concatenate([b2, jnp.full((CP - C,), -1e30, jnp.float32)])
    out = _mlp_tc(sums, w1s, b1, w2p, b2p)
    return out[:, :C]


# speedup vs baseline: 1.0335x; 1.0105x over previous
import functools

import jax
import jax.numpy as jnp
from jax import lax
from jax.experimental import pallas as pl
from jax.experimental.pallas import tpu as pltpu
from jax.experimental.pallas import tpu_sc as plsc

B, L, V, D, H, C = 16384, 50, 100000, 128, 256, 1000
CP = 1024
LANES = 16
NC, NS = 2, 16
NW = NC * NS
RPW = B // NW
RG = 8
NG = RPW // RG
DG = D // LANES
JU = 5


def _pool_sc(x, emb):
    mesh = plsc.VectorSubcoreMesh(core_axis_name="c", subcore_axis_name="s")

    @functools.partial(
        pl.kernel,
        out_type=jax.ShapeDtypeStruct((B, D), jnp.float32),
        mesh=mesh,
        scratch_types=[
            pltpu.VMEM((2, RG, L), jnp.int32),
            pltpu.VMEM((2, RG, L, D), jnp.float32),
            pltpu.VMEM((RG, D), jnp.float32),
            pltpu.SemaphoreType.DMA,
            pltpu.SemaphoreType.DMA,
            pltpu.SemaphoreType.DMA,
        ],
    )
    def pool(x_hbm, emb_hbm, out_hbm, idx_v, rows_v, out_v, sem0, sem1, isem):
        sems = (sem0, sem1)
        wid = lax.axis_index("s") * NC + lax.axis_index("c")
        row0 = wid * RPW

        def fire_idx(slot, g):
            r0 = row0 + g * RG
            pltpu.async_copy(x_hbm.at[pl.ds(r0, RG), :], idx_v.at[slot], isem)

        def wait_idx(slot, g):
            r0 = row0 + g * RG
            pltpu.make_async_copy(
                x_hbm.at[pl.ds(r0, RG), :], idx_v.at[slot], isem).wait()

        def fire_gather(slot, g):
            wait_idx(slot, g)
            for r in range(RG):
                pltpu.async_copy(
                    emb_hbm.at[idx_v.at[slot, r]], rows_v.at[slot, r], sems[slot]
                )

        def drain(slot):
            for r in range(RG):
                pltpu.make_async_copy(
                    emb_hbm.at[idx_v.at[slot, r]], rows_v.at[slot, r], sems[slot]
                ).wait()

        def reduce(slot, g):
            for r in range(RG):
                def body(jj, accs):
                    accs = list(accs)
                    for u in range(JU):
                        j = jj * JU + u
                        for d in range(DG):
                            accs[d] = accs[d] + rows_v[
                                slot, r, j, pl.ds(d * LANES, LANES)]
                    return tuple(accs)
                accs = lax.fori_loop(
                    0, L // JU, body,
                    tuple(jnp.zeros((LANES,), jnp.float32) for _ in range(DG)),
                )
                for d in range(DG):
                    out_v[r, pl.ds(d * LANES, LANES)] = accs[d]
            pltpu.sync_copy(out_v, out_hbm.at[pl.ds(row0 + g * RG, RG), :])

        fire_idx(0, 0)
        fire_gather(0, 0)
        fire_idx(1, 1)

        def body(k, carry):
            g = 2 * k
            fire_gather(1, g + 1)
            drain(0)
            fire_idx(0, g + 2)
            reduce(0, g)
            fire_gather(0, g + 2)
            drain(1)
            fire_idx(1, g + 3)
            reduce(1, g + 1)
            return carry

        lax.fori_loop(0, NG // 2 - 1, body, 0)
        fire_gather(1, NG - 1)
        drain(0)
        reduce(0, NG - 2)
        drain(1)
        reduce(1, NG - 1)

    return pool(x, emb)


def _mlp_tc(pooled, w1, b1, w2p, b2p):
    BM = 2048

    def body(p_ref, w1_ref, b1_ref, w2_ref, b2_ref, o_ref):
        h = jnp.dot(p_ref[...], w1_ref[...], preferred_element_type=jnp.float32)
        h = jnp.maximum(h + b1_ref[...], 0.0)
        logits = jnp.dot(h, w2_ref[...], preferred_element_type=jnp.float32)
        logits = logits + b2_ref[...]
        m = jnp.max(logits, axis=1, keepdims=True)
        z = logits - m
        o_ref[...] = z - jnp.log(jnp.sum(jnp.exp(z), axis=1, keepdims=True))

    return pl.pallas_call(
        body,
        grid=(B // BM,),
        in_specs=[
            pl.BlockSpec((BM, D), lambda i: (i, 0)),
            pl.BlockSpec((D, H), lambda i: (0, 0)),
            pl.BlockSpec((1, H), lambda i: (0, 0)),
            pl.BlockSpec((H, CP), lambda i: (0, 0)),
            pl.BlockSpec((1, CP), lambda i: (0, 0)),
        ],
        out_specs=pl.BlockSpec((BM, CP), lambda i: (i, 0)),
        out_shape=jax.ShapeDtypeStruct((B, CP), jnp.float32),
    )(pooled, w1, b1.reshape(1, H), w2p, b2p.reshape(1, CP))


def kernel(x, emb, w1, b1, w2, b2):
    sums = _pool_sc(x.astype(jnp.int32), emb)
    w1s = w1 * (1.0 / L)
    w2p = jnp.pad(w2, ((0, 0), (0, CP - C)))
    b2p = jnp.concatenate([b2, jnp.full((CP - C,), -1e30, jnp.float32)])
    out = _mlp_tc(sums, w1s, b1, w2p, b2p)
    return out[:, :C]
